# baseline (speedup 1.0000x reference)
import functools
import math

import numpy as np
import jax
import jax.numpy as jnp
from jax import lax
from jax.experimental import pallas as pl
from jax.experimental.pallas import tpu as pltpu
from jax.experimental.pallas import tpu_sc as plsc

D = 64
SEQ = 200
BW = 128
SCALE = 8.0


def _pos_embedding(max_len, d_model):
    pe = np.zeros((max_len, d_model), dtype=np.float32)
    position = np.arange(0, max_len, dtype=np.float32)[:, None]
    div_term = np.exp(-np.arange(0, d_model, 2, dtype=np.float32)
                      * (math.log(10000.0) / d_model))
    pe[:, 0::2] = np.sin(position * div_term)
    pe[:, 1::2] = np.cos(position * div_term)
    return pe


@functools.lru_cache(maxsize=None)
def _pe_flat_const(seq, d):
    return jnp.asarray(_pos_embedding(800, d)[:seq, :].reshape(-1))


def _make_body(batch):
    info = plsc.get_sparse_core_info()
    nc, ns = info.num_cores, info.num_subcores

    mesh = plsc.VectorSubcoreMesh(core_axis_name="c", subcore_axis_name="s")

    @functools.partial(
        pl.kernel,
        mesh=mesh,
        compiler_params=pltpu.CompilerParams(
            use_tc_tiling_on_sc=True, needs_layout_passes=False),
        out_type=jax.ShapeDtypeStruct((SEQ, D, batch), jnp.float32),
        scratch_types=[
            pltpu.VMEM((SEQ, BW), jnp.int32),
            pltpu.VMEM((SEQ, BW), jnp.int32),
            pltpu.VMEM((BW, BW), jnp.float32),
            pltpu.VMEM((BW, BW), jnp.float32),
            pltpu.VMEM((D, BW), jnp.float32),
            pltpu.VMEM((D, BW), jnp.float32),
            pltpu.VMEM((SEQ * D,), jnp.float32),
            pltpu.VMEM((D, 16), jnp.float32),
            pltpu.SemaphoreType.DMA,
            pltpu.SemaphoreType.DMA,
            pltpu.SemaphoreType.DMA,
            pltpu.SemaphoreType.DMA,
        ],
    )
    def body(table_hbm, xt_hbm, pe_hbm, out_hbm,
             phys_v, par_v, gath0, gath1, stag0, stag1, pe_v, peb_v,
             gsem0, gsem1, osem0, osem1):
        wid = lax.axis_index("s") * nc + lax.axis_index("c")
        col = wid * BW
        pltpu.sync_copy(pe_hbm, pe_v)
        pltpu.sync_copy(xt_hbm.at[:, pl.ds(col, BW)], phys_v)
        lanes = lax.iota(jnp.int32, 16)

        def prep(r, c2):
            for k in range(BW // 16):
                sl = pl.ds(k * 16, 16)
                v = phys_v[r, sl]
                phys_v[r, sl] = lax.shift_right_logical(v, 1)
                par_v[r, sl] = lax.shift_left(lax.bitwise_and(v, 1), 6)
            return c2

        lax.fori_loop(0, SEQ, prep, 0)

        def gather(s, gath, gsem):
            pltpu.make_async_copy(
                table_hbm.at[phys_v.at[s]], gath, gsem).start()

        def put(s, stag, osem):
            pltpu.make_async_copy(
                stag, out_hbm.at[s, :, pl.ds(col, BW)], osem).start()

        def wait_put(s, stag, osem):
            pltpu.make_async_copy(
                stag, out_hbm.at[s, :, pl.ds(col, BW)], osem).wait()

        def compute(s, gath, stag):
            base = jnp.full((16,), s * D, jnp.int32)

            def peb(u, c4):
                pevs = [plsc.load_gather(pe_v, [base + (4 * u + j)])
                        for j in range(4)]
                for j in range(4):
                    peb_v[4 * u + j, :] = pevs[j]
                return c4

            lax.fori_loop(0, D // 4, peb, 0)

            ni = BW // 16
            parv = [par_v[s, pl.ds(i0 * 16, 16)] for i0 in range(ni)]
            rowv = [lanes + i0 * 16 for i0 in range(ni)]

            def dstep(u, c5):
                d0 = 4 * u
                gs = [plsc.load_gather(gath, [rowv[i0], parv[i0] + (d0 + j)])
                      for j in range(4) for i0 in range(ni)]
                pevs = [peb_v[d0 + j, :] for j in range(4)]
                for j in range(4):
                    for i0 in range(ni):
                        stag[d0 + j, pl.ds(i0 * 16, 16)] = (
                            gs[j * ni + i0] * SCALE + pevs[j])
                return c5

            lax.fori_loop(0, D // 4, dstep, 0)

        gather(0, gath0, gsem0)

        def tstep(t, carry):
            s0 = 2 * t
            s1 = 2 * t + 1
            gather(s1, gath1, gsem1)
            pltpu.make_async_copy(
                table_hbm.at[phys_v.at[s0]], gath0, gsem0).wait()

            @pl.when(t > 0)
            def _():
                wait_put(s0 - 2, stag0, osem0)

            compute(s0, gath0, stag0)
            put(s0, stag0, osem0)

            @pl.when(t < SEQ // 2 - 1)
            def _():
                gather(s0 + 2, gath0, gsem0)

            pltpu.make_async_copy(
                table_hbm.at[phys_v.at[s1]], gath1, gsem1).wait()

            @pl.when(t > 0)
            def _():
                wait_put(s1 - 2, stag1, osem1)

            compute(s1, gath1, stag1)
            put(s1, stag1, osem1)
            return carry

        lax.fori_loop(0, SEQ // 2, tstep, 0)
        wait_put(SEQ - 2, stag0, osem0)
        wait_put(SEQ - 1, stag1, osem1)

    return body


def kernel(x, table):
    b, s = x.shape
    nv, d = table.shape
    table128 = table.reshape(nv // 2, 2 * d)
    xt = x.T
    pe = _pe_flat_const(s, d)
    body = _make_body(b)
    out_t = body(table128, xt, pe)
    return out_t.transpose(2, 0, 1)


# speedup vs baseline: 1.0158x; 1.0158x over previous
import functools
import math

import numpy as np
import jax
import jax.numpy as jnp
from jax import lax
from jax.experimental import pallas as pl
from jax.experimental.pallas import tpu as pltpu
from jax.experimental.pallas import tpu_sc as plsc

D = 64
SEQ = 200
BW = 128
SCALE = 8.0


def _pos_embedding(max_len, d_model):
    pe = np.zeros((max_len, d_model), dtype=np.float32)
    position = np.arange(0, max_len, dtype=np.float32)[:, None]
    div_term = np.exp(-np.arange(0, d_model, 2, dtype=np.float32)
                      * (math.log(10000.0) / d_model))
    pe[:, 0::2] = np.sin(position * div_term)
    pe[:, 1::2] = np.cos(position * div_term)
    return pe


@functools.lru_cache(maxsize=None)
def _pe_flat_const(seq, d):
    return jnp.asarray(_pos_embedding(800, d)[:seq, :].reshape(-1))


def _make_body(batch):
    info = plsc.get_sparse_core_info()
    nc, ns = info.num_cores, info.num_subcores

    mesh = plsc.VectorSubcoreMesh(core_axis_name="c", subcore_axis_name="s")

    @functools.partial(
        pl.kernel,
        mesh=mesh,
        compiler_params=pltpu.CompilerParams(
            use_tc_tiling_on_sc=True, needs_layout_passes=False),
        out_type=jax.ShapeDtypeStruct((SEQ, D, batch), jnp.float32),
        scratch_types=[
            pltpu.VMEM((SEQ, BW), jnp.int32),
            pltpu.VMEM((SEQ, BW), jnp.int32),
            pltpu.VMEM((BW, BW), jnp.float32),
            pltpu.VMEM((BW, BW), jnp.float32),
            pltpu.VMEM((D, BW), jnp.float32),
            pltpu.VMEM((D, BW), jnp.float32),
            pltpu.VMEM((SEQ * D,), jnp.float32),
            pltpu.VMEM((D, 16), jnp.float32),
            pltpu.SemaphoreType.DMA,
            pltpu.SemaphoreType.DMA,
            pltpu.SemaphoreType.DMA,
            pltpu.SemaphoreType.DMA,
        ],
    )
    def body(table_hbm, xt_hbm, pe_hbm, out_hbm,
             phys_v, par_v, gath0, gath1, stag0, stag1, pe_v, peb_v,
             gsem0, gsem1, osem0, osem1):
        wid = lax.axis_index("s") * nc + lax.axis_index("c")
        col = wid * BW
        pltpu.sync_copy(pe_hbm, pe_v)
        pltpu.sync_copy(xt_hbm.at[:, pl.ds(col, BW)], phys_v)
        lanes = lax.iota(jnp.int32, 16)

        def prep(r, c2):
            for k in range(BW // 16):
                sl = pl.ds(k * 16, 16)
                v = phys_v[r, sl]
                phys_v[r, sl] = lax.shift_right_logical(v, 1)
                par_v[r, sl] = lax.shift_left(lax.bitwise_and(v, 1), 6)
            return c2

        lax.fori_loop(0, SEQ, prep, 0)

        def gather(s, gath, gsem):
            pltpu.make_async_copy(
                table_hbm.at[phys_v.at[s]], gath, gsem).start()

        def put(s, stag, osem):
            pltpu.make_async_copy(
                stag, out_hbm.at[s, :, pl.ds(col, BW)], osem).start()

        def wait_put(s, stag, osem):
            pltpu.make_async_copy(
                stag, out_hbm.at[s, :, pl.ds(col, BW)], osem).wait()

        def compute(s, gath, stag):
            base = jnp.full((16,), s * D, jnp.int32)

            def peb(u, c4):
                pevs = [plsc.load_gather(pe_v, [base + (4 * u + j)])
                        for j in range(4)]
                for j in range(4):
                    peb_v[4 * u + j, :] = pevs[j]
                return c4

            lax.fori_loop(0, D // 4, peb, 0)

            ni = BW // 16
            parv = [par_v[s, pl.ds(i0 * 16, 16)] for i0 in range(ni)]
            rowv = [lanes + i0 * 16 for i0 in range(ni)]

            def dstep(u, c5):
                d0 = 2 * u
                gs = [plsc.load_gather(gath, [rowv[i0], parv[i0] + (d0 + j)])
                      for j in range(2) for i0 in range(ni)]
                pev0 = peb_v[d0, :]
                pev1 = peb_v[d0 + 1, :]
                for i0 in range(ni):
                    stag[d0, pl.ds(i0 * 16, 16)] = gs[i0] * SCALE + pev0
                for i0 in range(ni):
                    stag[d0 + 1, pl.ds(i0 * 16, 16)] = (
                        gs[ni + i0] * SCALE + pev1)
                return c5

            lax.fori_loop(0, D // 2, dstep, 0)

        gather(0, gath0, gsem0)

        def tstep(t, carry):
            s0 = 2 * t
            s1 = 2 * t + 1
            gather(s1, gath1, gsem1)
            pltpu.make_async_copy(
                table_hbm.at[phys_v.at[s0]], gath0, gsem0).wait()

            @pl.when(t > 0)
            def _():
                wait_put(s0 - 2, stag0, osem0)

            compute(s0, gath0, stag0)
            put(s0, stag0, osem0)

            @pl.when(t < SEQ // 2 - 1)
            def _():
                gather(s0 + 2, gath0, gsem0)

            pltpu.make_async_copy(
                table_hbm.at[phys_v.at[s1]], gath1, gsem1).wait()

            @pl.when(t > 0)
            def _():
                wait_put(s1 - 2, stag1, osem1)

            compute(s1, gath1, stag1)
            put(s1, stag1, osem1)
            return carry

        lax.fori_loop(0, SEQ // 2, tstep, 0)
        wait_put(SEQ - 2, stag0, osem0)
        wait_put(SEQ - 1, stag1, osem1)

    return body


def kernel(x, table):
    b, s = x.shape
    nv, d = table.shape
    table128 = table.reshape(nv // 2, 2 * d)
    xt = x.T
    pe = _pe_flat_const(s, d)
    body = _make_body(b)
    out_t = body(table128, xt, pe)
    return out_t.transpose(2, 0, 1)
